# baseline (speedup 1.0000x reference)
import functools
import math

import jax
import jax.numpy as jnp
from jax.experimental import pallas as pl
from jax.experimental.pallas import tpu as pltpu
from jax.experimental.pallas import tpu_sc as plsc

D_MODEL = 64
SCALE = math.sqrt(D_MODEL)
LANES = 16
WINDOW = 512


def _gather_scale(table, idx2d, n):
    mesh = plsc.VectorSubcoreMesh(core_axis_name="c", subcore_axis_name="s")

    @functools.partial(
        pl.kernel,
        out_type=jax.ShapeDtypeStruct((n, D_MODEL), jnp.float32),
        mesh=mesh,
        compiler_params=pltpu.CompilerParams(use_tc_tiling_on_sc=False),
    )
    def kern(table_hbm, i_hbm, o_hbm):
        def body(i_vmem, o_vmem):
            pltpu.sync_copy(table_hbm.at[i_vmem.at[0]], o_vmem)

            @pl.loop(0, WINDOW)
            def _(r):
                for c in range(0, D_MODEL, LANES):
                    slc = (pl.ds(r, 1), pl.ds(c, LANES))
                    o_vmem.at[*slc][...] = o_vmem.at[*slc][...] * SCALE

        pltpu.emit_pipeline(
            body,
            grid=(n // WINDOW,),
            in_specs=[pl.BlockSpec((1, WINDOW), lambda i: (0, i))],
            out_specs=[pl.BlockSpec((WINDOW, D_MODEL), lambda i: (i, 0))],
            core_axis_name=("c", "s"),
            dimension_semantics=(pltpu.PARALLEL,),
        )(i_hbm, o_hbm)

    return kern(table, idx2d)


def kernel(x, table):
    b, s = x.shape
    n = b * s
    idx2d = x.astype(jnp.int32).reshape(1, n)
    out = _gather_scale(table, idx2d, n)
    return out.reshape(b, s, D_MODEL)


# speedup vs baseline: 1.4728x; 1.4728x over previous
import functools
import math

import jax
import jax.numpy as jnp
from jax import lax
from jax.experimental import pallas as pl
from jax.experimental.pallas import tpu as pltpu
from jax.experimental.pallas import tpu_sc as plsc

D_MODEL = 64
SCALE = math.sqrt(D_MODEL)
PAD_W = 128

NUM_CORES = 2
NUM_SUBCORES = 16
NUM_WORKERS = NUM_CORES * NUM_SUBCORES

PREP_B = 1024
CHUNK = 256
LANES = 16


def _prep_table(table_t, vocab):
    grid = (vocab + PREP_B - 1) // PREP_B

    def body(t_ref, o_ref):
        o_ref[:, 0:D_MODEL] = t_ref[...].T * SCALE

    return pl.pallas_call(
        body,
        grid=(grid,),
        in_specs=[
            pl.BlockSpec((D_MODEL, PREP_B), lambda i: (0, i)),
        ],
        out_specs=pl.BlockSpec((PREP_B, PAD_W), lambda i: (i, 0)),
        out_shape=jax.ShapeDtypeStruct((vocab, PAD_W), jnp.float32),
    )(table_t)


def _gather64(t8, idx3d, n, nchunks):
    mesh = plsc.VectorSubcoreMesh(core_axis_name="c", subcore_axis_name="s")

    @functools.partial(
        pl.kernel,
        out_type=jax.ShapeDtypeStruct((n, PAD_W), jnp.float32),
        mesh=mesh,
        scratch_types=[
            pltpu.VMEM((nchunks * CHUNK,), jnp.int32),
            pltpu.VMEM((CHUNK, PAD_W), jnp.float32),
            pltpu.VMEM((CHUNK, PAD_W), jnp.float32),
            pltpu.SemaphoreType.DMA,
            pltpu.SemaphoreType.DMA,
        ],
    )
    def kern(t8_hbm, i_hbm, o_hbm, idx_v, g0, g1, sem0, sem1):
        wid = lax.axis_index("s") * NUM_CORES + lax.axis_index("c")
        per_w = nchunks * CHUNK
        base = wid * per_w

        pltpu.sync_copy(i_hbm.at[wid], idx_v)

        def start_gather(c, g, sem):
            pltpu.async_copy(t8_hbm.at[idx_v.at[pl.ds(c * CHUNK, CHUNK)]], g, sem)

        def wait_gather(c, g, sem):
            pltpu.make_async_copy(
                t8_hbm.at[idx_v.at[pl.ds(c * CHUNK, CHUNK)]], g, sem
            ).wait()

        def copy_out(c, g):
            pltpu.sync_copy(g, o_hbm.at[pl.ds(base + c * CHUNK, CHUNK)])

        start_gather(0, g0, sem0)
        start_gather(1, g1, sem1)

        @pl.loop(0, nchunks, step=2)
        def _(c):
            wait_gather(c, g0, sem0)
            copy_out(c, g0)

            @pl.when(c + 2 < nchunks)
            def _():
                start_gather(c + 2, g0, sem0)

            wait_gather(c + 1, g1, sem1)
            copy_out(c + 1, g1)

            @pl.when(c + 3 < nchunks)
            def _():
                start_gather(c + 3, g1, sem1)

    return kern(t8, idx3d)


def kernel(x, table):
    b, s = x.shape
    n = b * s
    vocab, d = table.shape
    nchunks = n // (NUM_WORKERS * CHUNK)
    idx3d = x.astype(jnp.int32).reshape(NUM_WORKERS, nchunks * CHUNK)
    t8 = _prep_table(jnp.transpose(table), vocab)
    out128 = _gather64(t8, idx3d, n, nchunks)
    return out128[:, 0:D_MODEL].reshape(b, s, D_MODEL)


# speedup vs baseline: 2.1389x; 1.4523x over previous
import functools
import math

import jax
import jax.numpy as jnp
from jax import lax
from jax.experimental import pallas as pl
from jax.experimental.pallas import tpu as pltpu
from jax.experimental.pallas import tpu_sc as plsc

D_MODEL = 64
SCALE = math.sqrt(D_MODEL)
PAD_W = 128

NUM_CORES = 2
NUM_SUBCORES = 16
NUM_WORKERS = NUM_CORES * NUM_SUBCORES

PREP_B = 4096
CHUNK = 256
LANES = 16


def _prep_table(table_t, vocab):
    grid = (vocab + PREP_B - 1) // PREP_B

    def body(t_ref, o_ref):
        o_ref[:, 0:D_MODEL] = t_ref[...].T * SCALE

    return pl.pallas_call(
        body,
        grid=(grid,),
        in_specs=[
            pl.BlockSpec((D_MODEL, PREP_B), lambda i: (0, i)),
        ],
        out_specs=pl.BlockSpec((PREP_B, PAD_W), lambda i: (i, 0)),
        out_shape=jax.ShapeDtypeStruct((vocab, PAD_W), jnp.float32),
        compiler_params=pltpu.CompilerParams(
            dimension_semantics=("parallel",)
        ),
    )(table_t)


def _gather64(t8, idx3d, n, nchunks):
    mesh = plsc.VectorSubcoreMesh(core_axis_name="c", subcore_axis_name="s")

    @functools.partial(
        pl.kernel,
        out_type=jax.ShapeDtypeStruct((n, PAD_W), jnp.float32),
        mesh=mesh,
        scratch_types=[
            pltpu.VMEM((nchunks * CHUNK,), jnp.int32),
            pltpu.VMEM((CHUNK, PAD_W), jnp.float32),
            pltpu.VMEM((CHUNK, PAD_W), jnp.float32),
            pltpu.SemaphoreType.DMA,
            pltpu.SemaphoreType.DMA,
        ],
    )
    def kern(t8_hbm, i_hbm, o_hbm, idx_v, g0, g1, sem0, sem1):
        wid = lax.axis_index("s") * NUM_CORES + lax.axis_index("c")
        per_w = nchunks * CHUNK
        base = wid * per_w

        pltpu.sync_copy(i_hbm.at[wid], idx_v)

        def start_gather(c, g, sem):
            pltpu.async_copy(t8_hbm.at[idx_v.at[pl.ds(c * CHUNK, CHUNK)]], g, sem)

        def wait_gather(c, g, sem):
            pltpu.make_async_copy(
                t8_hbm.at[idx_v.at[pl.ds(c * CHUNK, CHUNK)]], g, sem
            ).wait()

        def copy_out(c, g):
            pltpu.sync_copy(g, o_hbm.at[pl.ds(base + c * CHUNK, CHUNK)])

        start_gather(0, g0, sem0)
        start_gather(1, g1, sem1)

        @pl.loop(0, nchunks, step=2)
        def _(c):
            wait_gather(c, g0, sem0)
            copy_out(c, g0)

            @pl.when(c + 2 < nchunks)
            def _():
                start_gather(c + 2, g0, sem0)

            wait_gather(c + 1, g1, sem1)
            copy_out(c + 1, g1)

            @pl.when(c + 3 < nchunks)
            def _():
                start_gather(c + 3, g1, sem1)

    return kern(t8, idx3d)


def kernel(x, table):
    b, s = x.shape
    n = b * s
    vocab, d = table.shape
    nchunks = n // (NUM_WORKERS * CHUNK)
    idx3d = x.astype(jnp.int32).reshape(NUM_WORKERS, nchunks * CHUNK)
    t8 = _prep_table(jnp.transpose(table), vocab)
    out128 = _gather64(t8, idx3d, n, nchunks)
    return out128[:, 0:D_MODEL].reshape(b, s, D_MODEL)


# speedup vs baseline: 2.3406x; 1.0943x over previous
import functools
import math

import jax
import jax.numpy as jnp
from jax import lax
from jax.experimental import pallas as pl
from jax.experimental.pallas import tpu as pltpu
from jax.experimental.pallas import tpu_sc as plsc

D_MODEL = 64
SCALE = math.sqrt(D_MODEL)
PAD_W = 128

NUM_CORES = 2
NUM_SUBCORES = 16
NUM_WORKERS = NUM_CORES * NUM_SUBCORES

PREP_B = 8192
CHUNK = 320
LANES = 16


def _prep_table(table_t, vocab):
    grid = (vocab + PREP_B - 1) // PREP_B

    def body(t_ref, o_ref):
        o_ref[:, 0:D_MODEL] = t_ref[...].T * SCALE

    return pl.pallas_call(
        body,
        grid=(grid,),
        in_specs=[
            pl.BlockSpec((D_MODEL, PREP_B), lambda i: (0, i)),
        ],
        out_specs=pl.BlockSpec((PREP_B, PAD_W), lambda i: (i, 0)),
        out_shape=jax.ShapeDtypeStruct((vocab, PAD_W), jnp.float32),
        compiler_params=pltpu.CompilerParams(
            dimension_semantics=("parallel",)
        ),
    )(table_t)


def _gather64(t8, idx3d, n, nchunks):
    mesh = plsc.VectorSubcoreMesh(core_axis_name="c", subcore_axis_name="s")

    @functools.partial(
        pl.kernel,
        out_type=jax.ShapeDtypeStruct((n, PAD_W), jnp.float32),
        mesh=mesh,
        scratch_types=[
            pltpu.VMEM((nchunks * CHUNK,), jnp.int32),
            pltpu.VMEM((CHUNK, PAD_W), jnp.float32),
            pltpu.VMEM((CHUNK, PAD_W), jnp.float32),
            pltpu.SemaphoreType.DMA,
            pltpu.SemaphoreType.DMA,
        ],
    )
    def kern(t8_hbm, i_hbm, o_hbm, idx_v, g0, g1, sem0, sem1):
        wid = lax.axis_index("s") * NUM_CORES + lax.axis_index("c")
        per_w = nchunks * CHUNK
        base = wid * per_w

        pltpu.sync_copy(i_hbm.at[wid], idx_v)

        def start_gather(c, g, sem):
            pltpu.async_copy(t8_hbm.at[idx_v.at[pl.ds(c * CHUNK, CHUNK)]], g, sem)

        def wait_gather(c, g, sem):
            pltpu.make_async_copy(
                t8_hbm.at[idx_v.at[pl.ds(c * CHUNK, CHUNK)]], g, sem
            ).wait()

        def copy_out(c, g):
            pltpu.sync_copy(g, o_hbm.at[pl.ds(base + c * CHUNK, CHUNK)])

        start_gather(0, g0, sem0)
        start_gather(1, g1, sem1)

        @pl.loop(0, nchunks, step=2)
        def _(c):
            wait_gather(c, g0, sem0)
            copy_out(c, g0)

            @pl.when(c + 2 < nchunks)
            def _():
                start_gather(c + 2, g0, sem0)

            wait_gather(c + 1, g1, sem1)
            copy_out(c + 1, g1)

            @pl.when(c + 3 < nchunks)
            def _():
                start_gather(c + 3, g1, sem1)

    return kern(t8, idx3d)


def kernel(x, table):
    b, s = x.shape
    n = b * s
    vocab, d = table.shape
    nchunks = n // (NUM_WORKERS * CHUNK)
    idx3d = x.astype(jnp.int32).reshape(NUM_WORKERS, nchunks * CHUNK)
    t8 = _prep_table(jnp.transpose(table), vocab)
    out128 = _gather64(t8, idx3d, n, nchunks)
    return out128[:, 0:D_MODEL].reshape(b, s, D_MODEL)
